# baseline (speedup 1.0000x reference)
import functools
import math

import jax
import jax.numpy as jnp
from jax import lax
from jax.experimental import pallas as pl
from jax.experimental.pallas import tpu as pltpu
from jax.experimental.pallas import tpu_sc as plsc

N = 10000
E = 160000
D = 256
TAU = 1.0 / math.sqrt(D)

NC = 2
NS = 16
NW = NC * NS
L = 16

N_PAD = 10240
RPT = N_PAD // NW

E_PAD = 163840
E_ALLOC = E_PAD + 4096
EPT_A = E_PAD // NW
W_A = 64
NCH_A = EPT_A // W_A
W_B = 1024
NCH_B = E_PAD // W_B
SB = 64

_mesh = plsc.VectorSubcoreMesh(core_axis_name="c", subcore_axis_name="s")

_sc_params = pltpu.CompilerParams()
if "needs_layout_passes" in pltpu.CompilerParams.__dataclass_fields__:
    import dataclasses as _dataclasses

    _sc_params = _dataclasses.replace(_sc_params, needs_layout_passes=False)


def _proj_body(z_ref, wqt_ref, wkt_ref, wvt_ref, bq_ref, bk_ref, bv_ref,
               q_ref, k_ref, v_ref):
    zb = z_ref[...]
    q_ref[...] = (jnp.dot(zb, wqt_ref[...], preferred_element_type=jnp.float32)
                  + bq_ref[...]) * TAU
    k_ref[...] = (jnp.dot(zb, wkt_ref[...], preferred_element_type=jnp.float32)
                  + bk_ref[...])
    v_ref[...] = (jnp.dot(zb, wvt_ref[...], preferred_element_type=jnp.float32)
                  + bv_ref[...])


_ROWS_BLK = 2000

_proj = pl.pallas_call(
    _proj_body,
    grid=(N // _ROWS_BLK,),
    in_specs=[pl.BlockSpec((_ROWS_BLK, D), lambda i: (i, 0))]
    + [pl.BlockSpec((D, D), lambda i: (0, 0))] * 3
    + [pl.BlockSpec((1, D), lambda i: (0, 0))] * 3,
    out_specs=[pl.BlockSpec((_ROWS_BLK, D), lambda i: (i, 0))] * 3,
    out_shape=[jax.ShapeDtypeStruct((N, D), jnp.float32)] * 3,
)


@functools.partial(
    pl.kernel,
    out_type=[jax.ShapeDtypeStruct((E_ALLOC,), jnp.float32),
              jax.ShapeDtypeStruct((NW * N_PAD,), jnp.float32)],
    mesh=_mesh,
    compiler_params=_sc_params,
    scratch_types=[
        pltpu.VMEM((N_PAD,), jnp.float32),
        pltpu.VMEM((EPT_A + 2 * W_A,), jnp.int32),
        pltpu.VMEM((EPT_A + 2 * W_A,), jnp.int32),
        pltpu.VMEM((EPT_A,), jnp.float32),
        pltpu.VMEM((W_A, D), jnp.float32),
        pltpu.VMEM((W_A, D), jnp.float32),
        pltpu.VMEM((W_A, D), jnp.float32),
        pltpu.VMEM((W_A, D), jnp.float32),
        pltpu.SemaphoreType.DMA,
        pltpu.SemaphoreType.DMA,
    ],
)
def _edge_logits(k_hbm, q_hbm, src_hbm, dst_hbm, e_hbm, mpart_hbm,
                 m_part, srca, dsta, ebig, kb0, kb1, qb0, qb1, sem0, sem1):
    c = lax.axis_index("c")
    s = lax.axis_index("s")
    w = s * NC + c
    base = w * EPT_A
    lane = lax.iota(jnp.int32, L)

    neg = jnp.full((L,), -jnp.inf, jnp.float32)
    zi = jnp.zeros((L,), jnp.int32)

    @pl.loop(0, N_PAD, step=L)
    def _(i):
        m_part[pl.ds(i, L)] = neg

    pltpu.sync_copy(src_hbm.at[pl.ds(base, EPT_A)], srca.at[pl.ds(0, EPT_A)])
    pltpu.sync_copy(dst_hbm.at[pl.ds(base, EPT_A)], dsta.at[pl.ds(0, EPT_A)])

    @pl.loop(EPT_A, EPT_A + 2 * W_A, step=L)
    def _(i):
        srca[pl.ds(i, L)] = zi
        dsta[pl.ds(i, L)] = zi

    def _fire(off, kb, qb, sem):
        pltpu.make_async_copy(k_hbm.at[srca.at[pl.ds(off, W_A)]], kb,
                              sem).start()
        pltpu.make_async_copy(q_hbm.at[dsta.at[pl.ds(off, W_A)]], qb,
                              sem).start()

    def _wait(off, kb, qb, sem):
        pltpu.make_async_copy(k_hbm.at[srca.at[pl.ds(off, W_A)]], kb,
                              sem).wait()
        pltpu.make_async_copy(q_hbm.at[dsta.at[pl.ds(off, W_A)]], qb,
                              sem).wait()

    def _process(off, kb, qb):
        @pl.loop(0, W_A, step=L)
        def _(g):
            ev = jnp.zeros((L,), jnp.float32)
            for j in range(L):
                i = g + j
                acc = kb[i, pl.ds(0, L)] * qb[i, pl.ds(0, L)]
                for t in range(1, D // L):
                    acc = acc + kb[i, pl.ds(t * L, L)] * qb[i, pl.ds(t * L, L)]
                ev = jnp.where(lane == j, jnp.sum(acc), ev)
            ebig[pl.ds(off + g, L)] = ev
            dvec = dsta[pl.ds(off + g, L)]

            def _mbody(_):
                cur = plsc.load_gather(m_part, [dvec])
                pend = cur < ev
                plsc.store_scatter(m_part, [dvec], jnp.maximum(cur, ev),
                                   mask=pend)
                return jnp.any(pend)

            lax.while_loop(lambda go: go, _mbody, jnp.bool_(True))

    _fire(0, kb0, qb0, sem0)

    @pl.loop(0, NCH_A, step=2)
    def _(ci):
        off0 = ci * W_A
        _fire(off0 + W_A, kb1, qb1, sem1)
        _wait(off0, kb0, qb0, sem0)
        _process(off0, kb0, qb0)
        _fire(off0 + 2 * W_A, kb0, qb0, sem0)
        _wait(off0 + W_A, kb1, qb1, sem1)
        _process(off0 + W_A, kb1, qb1)

    _wait(NCH_A * W_A, kb0, qb0, sem0)

    pltpu.sync_copy(ebig, e_hbm.at[pl.ds(base, EPT_A)])
    pltpu.sync_copy(m_part, mpart_hbm.at[pl.ds(w * N_PAD, N_PAD)])


@functools.partial(
    pl.kernel,
    out_type=jax.ShapeDtypeStruct((N_PAD, D), jnp.float32),
    mesh=_mesh,
    compiler_params=_sc_params,
    scratch_types=[
        pltpu.VMEM((RPT, D), jnp.float32),
        pltpu.VMEM((RPT,), jnp.float32),
        pltpu.VMEM((RPT,), jnp.float32),
        pltpu.VMEM((W_B,), jnp.int32),
        pltpu.VMEM((W_B,), jnp.int32),
        pltpu.VMEM((W_B,), jnp.int32),
        pltpu.VMEM((W_B,), jnp.int32),
        pltpu.VMEM((W_B,), jnp.float32),
        pltpu.VMEM((W_B,), jnp.float32),
        pltpu.VMEM((W_B,), jnp.int32),
        pltpu.VMEM((W_B,), jnp.int32),
        pltpu.VMEM((W_B,), jnp.float32),
        pltpu.VMEM((SB, D), jnp.float32),
        pltpu.VMEM((SB, D), jnp.float32),
        pltpu.SemaphoreType.DMA,
        pltpu.SemaphoreType.DMA,
        pltpu.SemaphoreType.DMA,
        pltpu.SemaphoreType.DMA,
    ],
)
def _aggregate(v_hbm, src_hbm, dst_hbm, e_hbm, mpart_hbm, h_hbm,
               h_own, den_own, m_own, srcb0, srcb1, dstb0, dstb1, eb0, eb1,
               psrc, pdst, pe, vg0, vg1, sem0, sem1, semv0, semv1):
    c = lax.axis_index("c")
    s = lax.axis_index("s")
    w = s * NC + c
    r0 = w * RPT
    lane = lax.iota(jnp.int32, L)
    zv = jnp.zeros((L,), jnp.float32)
    zi = jnp.zeros((L,), jnp.int32)

    pltpu.sync_copy(mpart_hbm.at[pl.ds(pl.multiple_of(r0, 8), RPT)], m_own)

    @pl.loop(1, NW)
    def _(p):
        poff = pl.multiple_of(p * N_PAD + r0, 8)
        pltpu.sync_copy(mpart_hbm.at[pl.ds(poff, RPT)], pe.at[pl.ds(0, RPT)])

        @pl.loop(0, RPT, step=L)
        def _(i):
            m_own[pl.ds(i, L)] = jnp.maximum(m_own[pl.ds(i, L)],
                                             pe[pl.ds(i, L)])

    @pl.loop(0, RPT)
    def _(r):
        for t in range(D // L):
            h_own[r, pl.ds(t * L, L)] = zv

    @pl.loop(0, RPT, step=L)
    def _(i):
        den_own[pl.ds(i, L)] = zv

    @pl.loop(0, W_B, step=L)
    def _(i):
        psrc[pl.ds(i, L)] = zi
        pdst[pl.ds(i, L)] = zi

    def _fire(off, srcb, dstb, eb, sem):
        pltpu.make_async_copy(src_hbm.at[pl.ds(off, W_B)], srcb, sem).start()
        pltpu.make_async_copy(dst_hbm.at[pl.ds(off, W_B)], dstb, sem).start()
        pltpu.make_async_copy(e_hbm.at[pl.ds(off, W_B)], eb, sem).start()

    def _wait(off, srcb, dstb, eb, sem):
        pltpu.make_async_copy(src_hbm.at[pl.ds(off, W_B)], srcb, sem).wait()
        pltpu.make_async_copy(dst_hbm.at[pl.ds(off, W_B)], dstb, sem).wait()
        pltpu.make_async_copy(e_hbm.at[pl.ds(off, W_B)], eb, sem).wait()

    def _vfire(sb, vg, sem):
        pltpu.make_async_copy(v_hbm.at[psrc.at[pl.ds(sb * SB, SB)]], vg,
                              sem).start()

    def _vwait(sb, vg, sem):
        pltpu.make_async_copy(v_hbm.at[psrc.at[pl.ds(sb * SB, SB)]], vg,
                              sem).wait()

    def _accum(sb, vg, np_):
        @pl.loop(0, SB // L)
        def _(gj):
            gbase = sb * SB + gj * L
            dlg = pdst[pl.ds(gbase, L)]
            eg = pe[pl.ds(gbase, L)]
            validv = (gbase + lane) < np_
            dls = jnp.where(validv, dlg, 0)
            mg = plsc.load_gather(m_own, [dls])
            exv = jnp.where(validv, jnp.exp(eg - mg), 0.0)
            for j in range(L):
                dj = dls[j]
                exj = exv[j]
                vrow = gj * L + j
                for t in range(D // L):
                    plsc.addupdate(h_own.at[dj, pl.ds(t * L, L)],
                                   vg[vrow, pl.ds(t * L, L)] * exj)
                plsc.addupdate_scatter(
                    den_own, [jnp.full_like(lane, dj)],
                    jnp.where(lane == 0, exj, 0.0), mask=lane == 0)

    def _process(srcb, dstb, eb):
        def _compact(gi, np_):
            gg = gi * L
            dl = dstb[pl.ds(gg, L)] - r0
            valid = (dl >= 0) & (dl < RPT)
            plsc.store_compressed(psrc.at[pl.ds(np_, L)],
                                  srcb[pl.ds(gg, L)], mask=valid)
            plsc.store_compressed(pdst.at[pl.ds(np_, L)], dl, mask=valid)
            plsc.store_compressed(pe.at[pl.ds(np_, L)],
                                  eb[pl.ds(gg, L)], mask=valid)
            cnt = plsc.all_reduce_population_count(valid)
            return np_ + cnt[0]

        np_ = lax.fori_loop(0, W_B // L, _compact, jnp.int32(0))
        nsb = lax.shift_right_logical(np_ + (SB - 1), 6)

        @pl.when(nsb > 0)
        def _():
            _vfire(0, vg0, semv0)

        @pl.loop(0, nsb, step=2)
        def _(b):
            @pl.when(b + 1 < nsb)
            def _():
                _vfire(b + 1, vg1, semv1)

            _vwait(b, vg0, semv0)
            _accum(b, vg0, np_)

            @pl.when(b + 2 < nsb)
            def _():
                _vfire(b + 2, vg0, semv0)

            @pl.when(b + 1 < nsb)
            def _():
                _vwait(b + 1, vg1, semv1)
                _accum(b + 1, vg1, np_)

    _fire(0, srcb0, dstb0, eb0, sem0)

    @pl.loop(0, NCH_B, step=2)
    def _(ci):
        off0 = ci * W_B
        _fire(off0 + W_B, srcb1, dstb1, eb1, sem1)
        _wait(off0, srcb0, dstb0, eb0, sem0)
        _process(srcb0, dstb0, eb0)
        _fire(off0 + 2 * W_B, srcb0, dstb0, eb0, sem0)
        _wait(off0 + W_B, srcb1, dstb1, eb1, sem1)
        _process(srcb1, dstb1, eb1)

    _wait(NCH_B * W_B, srcb0, dstb0, eb0, sem0)

    @pl.loop(0, RPT, step=16)
    def _(r):
        denv = den_own[pl.ds(r, 16)]
        scale16 = jnp.where(denv > 0, 1.0 / jnp.where(denv > 0, denv, 1.0),
                            0.0)
        for j in range(16):
            scale = scale16[j]
            for t in range(D // L):
                h_own[r + j, pl.ds(t * L, L)] = (h_own[r + j, pl.ds(t * L, L)]
                                                 * scale)
        pltpu.sync_copy(h_own.at[pl.ds(r, 16)], h_hbm.at[pl.ds(r0 + r, 16)])


def kernel(z, edge_index, Wq, bq, Wk, bk, Wv, bv):
    ei = edge_index.astype(jnp.int32)
    src, dst = ei[0], ei[1]
    pad = E_ALLOC - E
    src_p = jnp.concatenate([src, jnp.zeros((pad,), jnp.int32)])
    dst_p = jnp.concatenate([dst, jnp.full((pad,), N_PAD - 1, jnp.int32)])
    q, k, v = _proj(z, Wq.T, Wk.T, Wv.T,
                    bq.reshape(1, D), bk.reshape(1, D), bv.reshape(1, D))
    e, mpart = _edge_logits(k, q, src_p, dst_p)
    h = _aggregate(v, src_p, dst_p, e, mpart)
    return h[:N]


# speedup vs baseline: 3.2175x; 3.2175x over previous
import functools
import math

import jax
import jax.numpy as jnp
from jax import lax
from jax.experimental import pallas as pl
from jax.experimental.pallas import tpu as pltpu
from jax.experimental.pallas import tpu_sc as plsc

N = 10000
E = 160000
D = 256
TAU = 1.0 / math.sqrt(D)

NC = 2
NS = 16
NW = NC * NS
L = 16

N_PAD = 10240
RPT = N_PAD // NW

E_PAD = 163840
E_ALLOC = E_PAD + 4096
EPT_A = E_PAD // NW
W_A = 64
NCH_A = EPT_A // W_A
W_B = 1024
NCH_B = E_PAD // W_B
SB = 48

_mesh = plsc.VectorSubcoreMesh(core_axis_name="c", subcore_axis_name="s")

_sc_params = pltpu.CompilerParams()
if "needs_layout_passes" in pltpu.CompilerParams.__dataclass_fields__:
    import dataclasses as _dataclasses

    _sc_params = _dataclasses.replace(_sc_params, needs_layout_passes=False)


def _proj_body(z_ref, wqt_ref, wkt_ref, wvt_ref, bq_ref, bk_ref, bv_ref,
               q_ref, k_ref, v_ref):
    zb = z_ref[...]
    q_ref[...] = (jnp.dot(zb, wqt_ref[...], preferred_element_type=jnp.float32)
                  + bq_ref[...]) * TAU
    k_ref[...] = (jnp.dot(zb, wkt_ref[...], preferred_element_type=jnp.float32)
                  + bk_ref[...])
    v_ref[...] = (jnp.dot(zb, wvt_ref[...], preferred_element_type=jnp.float32)
                  + bv_ref[...])


_ROWS_BLK = 2000

_proj = pl.pallas_call(
    _proj_body,
    grid=(N // _ROWS_BLK,),
    in_specs=[pl.BlockSpec((_ROWS_BLK, D), lambda i: (i, 0))]
    + [pl.BlockSpec((D, D), lambda i: (0, 0))] * 3
    + [pl.BlockSpec((1, D), lambda i: (0, 0))] * 3,
    out_specs=[pl.BlockSpec((_ROWS_BLK, D), lambda i: (i, 0))] * 3,
    out_shape=[jax.ShapeDtypeStruct((N, D), jnp.float32)] * 3,
)


@functools.partial(
    pl.kernel,
    out_type=[jax.ShapeDtypeStruct((E_ALLOC,), jnp.float32),
              jax.ShapeDtypeStruct((NW * N_PAD,), jnp.float32)],
    mesh=_mesh,
    compiler_params=_sc_params,
    scratch_types=[
        pltpu.VMEM((N_PAD,), jnp.float32),
        pltpu.VMEM((EPT_A + 2 * W_A,), jnp.int32),
        pltpu.VMEM((EPT_A + 2 * W_A,), jnp.int32),
        pltpu.VMEM((EPT_A,), jnp.float32),
        pltpu.VMEM((W_A, D), jnp.float32),
        pltpu.VMEM((W_A, D), jnp.float32),
        pltpu.VMEM((W_A, D), jnp.float32),
        pltpu.VMEM((W_A, D), jnp.float32),
        pltpu.SemaphoreType.DMA,
        pltpu.SemaphoreType.DMA,
    ],
)
def _edge_logits(k_hbm, q_hbm, src_hbm, dst_hbm, e_hbm, mpart_hbm,
                 m_part, srca, dsta, ebig, kb0, kb1, qb0, qb1, sem0, sem1):
    c = lax.axis_index("c")
    s = lax.axis_index("s")
    w = s * NC + c
    base = w * EPT_A
    lane = lax.iota(jnp.int32, L)

    neg = jnp.full((L,), -jnp.inf, jnp.float32)
    zi = jnp.zeros((L,), jnp.int32)

    @pl.loop(0, N_PAD, step=L)
    def _(i):
        m_part[pl.ds(i, L)] = neg

    pltpu.sync_copy(src_hbm.at[pl.ds(base, EPT_A)], srca.at[pl.ds(0, EPT_A)])
    pltpu.sync_copy(dst_hbm.at[pl.ds(base, EPT_A)], dsta.at[pl.ds(0, EPT_A)])

    @pl.loop(EPT_A, EPT_A + 2 * W_A, step=L)
    def _(i):
        srca[pl.ds(i, L)] = zi
        dsta[pl.ds(i, L)] = zi

    def _fire(off, kb, qb, sem):
        pltpu.make_async_copy(k_hbm.at[srca.at[pl.ds(off, W_A)]], kb,
                              sem).start()
        pltpu.make_async_copy(q_hbm.at[dsta.at[pl.ds(off, W_A)]], qb,
                              sem).start()

    def _wait(off, kb, qb, sem):
        pltpu.make_async_copy(k_hbm.at[srca.at[pl.ds(off, W_A)]], kb,
                              sem).wait()
        pltpu.make_async_copy(q_hbm.at[dsta.at[pl.ds(off, W_A)]], qb,
                              sem).wait()

    def _process(off, kb, qb):
        @pl.loop(0, W_A, step=L)
        def _(g):
            ev = jnp.zeros((L,), jnp.float32)
            for j in range(L):
                i = g + j
                acc = kb[i, pl.ds(0, L)] * qb[i, pl.ds(0, L)]
                for t in range(1, D // L):
                    acc = acc + kb[i, pl.ds(t * L, L)] * qb[i, pl.ds(t * L, L)]
                ev = jnp.where(lane == j, jnp.sum(acc), ev)
            ebig[pl.ds(off + g, L)] = ev
            dvec = dsta[pl.ds(off + g, L)]

            def _mbody(_):
                cur = plsc.load_gather(m_part, [dvec])
                pend = cur < ev
                plsc.store_scatter(m_part, [dvec], jnp.maximum(cur, ev),
                                   mask=pend)
                return jnp.any(pend)

            lax.while_loop(lambda go: go, _mbody, jnp.bool_(True))

    _fire(0, kb0, qb0, sem0)

    @pl.loop(0, NCH_A, step=2)
    def _(ci):
        off0 = ci * W_A
        _fire(off0 + W_A, kb1, qb1, sem1)
        _wait(off0, kb0, qb0, sem0)
        _process(off0, kb0, qb0)
        _fire(off0 + 2 * W_A, kb0, qb0, sem0)
        _wait(off0 + W_A, kb1, qb1, sem1)
        _process(off0 + W_A, kb1, qb1)

    _wait(NCH_A * W_A, kb0, qb0, sem0)

    pltpu.sync_copy(ebig, e_hbm.at[pl.ds(base, EPT_A)])
    pltpu.sync_copy(m_part, mpart_hbm.at[pl.ds(w * N_PAD, N_PAD)])


@functools.partial(
    pl.kernel,
    out_type=jax.ShapeDtypeStruct((N_PAD, D), jnp.float32),
    mesh=_mesh,
    compiler_params=_sc_params,
    scratch_types=[
        pltpu.VMEM((RPT, D), jnp.float32),
        pltpu.VMEM((RPT,), jnp.float32),
        pltpu.VMEM((RPT,), jnp.float32),
        pltpu.VMEM((RPT,), jnp.float32),
        pltpu.VMEM((W_B,), jnp.int32),
        pltpu.VMEM((W_B,), jnp.int32),
        pltpu.VMEM((W_B,), jnp.int32),
        pltpu.VMEM((W_B,), jnp.int32),
        pltpu.VMEM((W_B,), jnp.int32),
        pltpu.VMEM((W_B,), jnp.int32),
        pltpu.VMEM((W_B,), jnp.int32),
        pltpu.VMEM((W_B,), jnp.int32),
        pltpu.VMEM((W_B,), jnp.float32),
        pltpu.VMEM((W_B,), jnp.float32),
        pltpu.VMEM((W_B,), jnp.float32),
        pltpu.VMEM((W_B,), jnp.float32),
        pltpu.VMEM((SB, D), jnp.float32),
        pltpu.VMEM((SB, D), jnp.float32),
        pltpu.SemaphoreType.DMA,
        pltpu.SemaphoreType.DMA,
        pltpu.SemaphoreType.DMA,
        pltpu.SemaphoreType.DMA,
        pltpu.SemaphoreType.DMA,
        pltpu.SemaphoreType.DMA,
    ],
)
def _aggregate(v_hbm, src_hbm, dst_hbm, e_hbm, mpart_hbm, h_hbm,
               h_own, den_own, m_own, m_stage,
               sb0, sb1, sb2, sb3, db0, db1, db2, db3, ebf0, ebf1, ebf2,
               ebf3, vg0, vg1, sm0, sm1, sm2, sm3, smv0, smv1):
    c = lax.axis_index("c")
    s = lax.axis_index("s")
    w = s * NC + c
    r0 = w * RPT
    lane = lax.iota(jnp.int32, L)
    zv = jnp.zeros((L,), jnp.float32)

    pltpu.sync_copy(mpart_hbm.at[pl.ds(pl.multiple_of(r0, 8), RPT)], m_own)

    @pl.loop(1, NW)
    def _(p):
        poff = pl.multiple_of(p * N_PAD + r0, 8)
        pltpu.sync_copy(mpart_hbm.at[pl.ds(poff, RPT)], m_stage)

        @pl.loop(0, RPT, step=L)
        def _(i):
            m_own[pl.ds(i, L)] = jnp.maximum(m_own[pl.ds(i, L)],
                                             m_stage[pl.ds(i, L)])

    @pl.loop(0, RPT)
    def _(r):
        for t in range(D // L):
            h_own[r, pl.ds(t * L, L)] = zv

    @pl.loop(0, RPT, step=L)
    def _(i):
        den_own[pl.ds(i, L)] = zv

    def _sfire(ch, sb, db, eb, sem):
        off = ch * W_B
        pltpu.make_async_copy(src_hbm.at[pl.ds(off, W_B)], sb, sem).start()
        pltpu.make_async_copy(dst_hbm.at[pl.ds(off, W_B)], db, sem).start()
        pltpu.make_async_copy(e_hbm.at[pl.ds(off, W_B)], eb, sem).start()

    def _swait(ch, sb, db, eb, sem):
        off = ch * W_B
        pltpu.make_async_copy(src_hbm.at[pl.ds(off, W_B)], sb, sem).wait()
        pltpu.make_async_copy(dst_hbm.at[pl.ds(off, W_B)], db, sem).wait()
        pltpu.make_async_copy(e_hbm.at[pl.ds(off, W_B)], eb, sem).wait()

    def _compact(sb, db, eb):
        def _body(gi, np_):
            gg = gi * L
            dl = db[pl.ds(gg, L)] - r0
            valid = (dl >= 0) & (dl < RPT)
            plsc.store_compressed(sb.at[pl.ds(np_, L)],
                                  sb[pl.ds(gg, L)], mask=valid)
            plsc.store_compressed(db.at[pl.ds(np_, L)], dl, mask=valid)
            plsc.store_compressed(eb.at[pl.ds(np_, L)],
                                  eb[pl.ds(gg, L)], mask=valid)
            cnt = plsc.all_reduce_population_count(valid)
            return np_ + cnt[0]

        return lax.fori_loop(0, W_B // L, _body, jnp.int32(0))

    def _vfire(sb, vg, sem, np_):
        ng16 = jnp.minimum(lax.shift_right_logical(np_ + (L - 1), 4),
                           SB // L)

        @pl.loop(0, ng16)
        def _(g):
            pltpu.make_async_copy(v_hbm.at[sb.at[pl.ds(g * L, L)]],
                                  vg.at[pl.ds(g * L, L)], sem).start()

    def _vwait(sb, vg, sem, np_):
        ng16 = jnp.minimum(lax.shift_right_logical(np_ + (L - 1), 4),
                           SB // L)

        @pl.loop(0, ng16)
        def _(g):
            pltpu.make_async_copy(v_hbm.at[sb.at[pl.ds(g * L, L)]],
                                  vg.at[pl.ds(g * L, L)], sem).wait()

    def _accum_group(db, eb, vg, gj, vbase, np_):
        gbase = gj * L
        dlg = db[pl.ds(gbase, L)]
        eg = eb[pl.ds(gbase, L)]
        validv = (gbase + lane) < np_
        dls = jnp.where(validv, dlg, 0)
        mg = plsc.load_gather(m_own, [dls])
        exv = jnp.where(validv, jnp.exp(eg - mg), 0.0)
        for j in range(L):
            dj = dls[j]
            exj = exv[j]
            vrow = vbase + j
            for t in range(D // L):
                plsc.addupdate(h_own.at[dj, pl.ds(t * L, L)],
                               vg[vrow, pl.ds(t * L, L)] * exj)
            plsc.addupdate_scatter(
                den_own, [jnp.full_like(lane, dj)],
                jnp.where(lane == 0, exj, 0.0), mask=lane == 0)

    def _accum(sb, db, eb, vg, np_):
        ngj = lax.shift_right_logical(np_ + (L - 1), 4)

        @pl.loop(0, ngj)
        def _(gj):
            inbuf = gj < (SB // L)

            @pl.when(jnp.logical_not(inbuf))
            def _():
                pltpu.sync_copy(v_hbm.at[sb.at[pl.ds(gj * L, L)]],
                                vg.at[pl.ds(0, L)])

            vbase = jnp.where(inbuf, gj * L, 0)
            _accum_group(db, eb, vg, gj, vbase, np_)

    _ring = [(sb0, db0, ebf0, sm0), (sb1, db1, ebf1, sm1),
             (sb2, db2, ebf2, sm2), (sb3, db3, ebf3, sm3)]
    _vring = [(vg0, smv0), (vg1, smv1)]

    _sfire(0, sb0, db0, ebf0, sm0)
    _sfire(1, sb1, db1, ebf1, sm1)
    _sfire(2, sb2, db2, ebf2, sm2)

    def _stage(k, np_prev, pos):
        sbk, dbk, ebk, smk = _ring[pos % 4]
        vgk, smvk = _vring[pos % 2]
        sbp, dbp, ebp, smp = _ring[(pos - 1) % 4]
        vgp, smvp = _vring[(pos - 1) % 2]
        _swait(k, sbk, dbk, ebk, smk)
        np_k = _compact(sbk, dbk, ebk)
        _vfire(sbk, vgk, smvk, np_k)
        _vwait(sbp, vgp, smvp, np_prev)
        _accum(sbp, dbp, ebp, vgp, np_prev)

        @pl.when(k + 3 < NCH_B)
        def _():
            _sfire(k + 3, sbp, dbp, ebp, smp)

        return np_k

    def _quad(it, np_carry):
        k = it * 4
        np_carry = _stage(k, np_carry, 0)
        np_carry = _stage(k + 1, np_carry, 1)
        np_carry = _stage(k + 2, np_carry, 2)
        np_carry = _stage(k + 3, np_carry, 3)
        return np_carry

    np_last = lax.fori_loop(0, NCH_B // 4, _quad, jnp.int32(0))

    _vwait(sb3, vg1, smv1, np_last)
    _accum(sb3, db3, ebf3, vg1, np_last)

    @pl.loop(0, RPT, step=16)
    def _(r):
        denv = den_own[pl.ds(r, 16)]
        scale16 = jnp.where(denv > 0, 1.0 / jnp.where(denv > 0, denv, 1.0),
                            0.0)
        for j in range(16):
            scale = scale16[j]
            for t in range(D // L):
                h_own[r + j, pl.ds(t * L, L)] = (h_own[r + j, pl.ds(t * L, L)]
                                                 * scale)
        pltpu.sync_copy(h_own.at[pl.ds(r, 16)], h_hbm.at[pl.ds(r0 + r, 16)])


def kernel(z, edge_index, Wq, bq, Wk, bk, Wv, bv):
    ei = edge_index.astype(jnp.int32)
    src, dst = ei[0], ei[1]
    pad = E_ALLOC - E
    src_p = jnp.concatenate([src, jnp.zeros((pad,), jnp.int32)])
    dst_p = jnp.concatenate([dst, jnp.full((pad,), N_PAD - 1, jnp.int32)])
    q, k, v = _proj(z, Wq.T, Wk.T, Wv.T,
                    bq.reshape(1, D), bk.reshape(1, D), bv.reshape(1, D))
    e, mpart = _edge_logits(k, q, src_p, dst_p)
    h = _aggregate(v, src_p, dst_p, e, mpart)
    return h[:N]


# speedup vs baseline: 3.2592x; 1.0130x over previous
import functools
import math

import jax
import jax.numpy as jnp
from jax import lax
from jax.experimental import pallas as pl
from jax.experimental.pallas import tpu as pltpu
from jax.experimental.pallas import tpu_sc as plsc

N = 10000
E = 160000
D = 256
TAU = 1.0 / math.sqrt(D)

NC = 2
NS = 16
NW = NC * NS
L = 16

N_PAD = 10240
RPT = N_PAD // NW

E_PAD = 163840
E_ALLOC = E_PAD + 4096
EPT_A = E_PAD // NW
W_A = 64
NCH_A = EPT_A // W_A
W_B = 1024
NCH_B = E_PAD // W_B
SB = 48

_mesh = plsc.VectorSubcoreMesh(core_axis_name="c", subcore_axis_name="s")

_sc_params = pltpu.CompilerParams()
if "needs_layout_passes" in pltpu.CompilerParams.__dataclass_fields__:
    import dataclasses as _dataclasses

    _sc_params = _dataclasses.replace(_sc_params, needs_layout_passes=False)


def _proj_body(z_ref, wqt_ref, wkt_ref, wvt_ref, bq_ref, bk_ref, bv_ref,
               q_ref, k_ref, v_ref):
    zb = z_ref[...]
    q_ref[...] = (jnp.dot(zb, wqt_ref[...], preferred_element_type=jnp.float32)
                  + bq_ref[...]) * TAU
    k_ref[...] = (jnp.dot(zb, wkt_ref[...], preferred_element_type=jnp.float32)
                  + bk_ref[...])
    v_ref[...] = (jnp.dot(zb, wvt_ref[...], preferred_element_type=jnp.float32)
                  + bv_ref[...])


_ROWS_BLK = 2000

_proj = pl.pallas_call(
    _proj_body,
    grid=(N // _ROWS_BLK,),
    in_specs=[pl.BlockSpec((_ROWS_BLK, D), lambda i: (i, 0))]
    + [pl.BlockSpec((D, D), lambda i: (0, 0))] * 3
    + [pl.BlockSpec((1, D), lambda i: (0, 0))] * 3,
    out_specs=[pl.BlockSpec((_ROWS_BLK, D), lambda i: (i, 0))] * 3,
    out_shape=[jax.ShapeDtypeStruct((N, D), jnp.float32)] * 3,
)


@functools.partial(
    pl.kernel,
    out_type=[jax.ShapeDtypeStruct((E_ALLOC,), jnp.float32),
              jax.ShapeDtypeStruct((NW * N_PAD,), jnp.float32)],
    mesh=_mesh,
    compiler_params=_sc_params,
    scratch_types=[
        pltpu.VMEM((N_PAD,), jnp.float32),
        pltpu.VMEM((EPT_A + 2 * W_A,), jnp.int32),
        pltpu.VMEM((EPT_A + 2 * W_A,), jnp.int32),
        pltpu.VMEM((EPT_A,), jnp.float32),
        pltpu.VMEM((W_A, D), jnp.float32),
        pltpu.VMEM((W_A, D), jnp.float32),
        pltpu.VMEM((W_A, D), jnp.float32),
        pltpu.VMEM((W_A, D), jnp.float32),
        pltpu.SemaphoreType.DMA,
        pltpu.SemaphoreType.DMA,
    ],
)
def _edge_logits(k_hbm, q_hbm, src_hbm, dst_hbm, e_hbm, mpart_hbm,
                 m_part, srca, dsta, ebig, kb0, kb1, qb0, qb1, sem0, sem1):
    c = lax.axis_index("c")
    s = lax.axis_index("s")
    w = s * NC + c
    base = w * EPT_A
    lane = lax.iota(jnp.int32, L)

    neg = jnp.full((L,), -jnp.inf, jnp.float32)
    zi = jnp.zeros((L,), jnp.int32)

    @pl.loop(0, N_PAD, step=L)
    def _(i):
        m_part[pl.ds(i, L)] = neg

    pltpu.sync_copy(src_hbm.at[pl.ds(base, EPT_A)], srca.at[pl.ds(0, EPT_A)])
    pltpu.sync_copy(dst_hbm.at[pl.ds(base, EPT_A)], dsta.at[pl.ds(0, EPT_A)])

    @pl.loop(EPT_A, EPT_A + 2 * W_A, step=L)
    def _(i):
        srca[pl.ds(i, L)] = zi
        dsta[pl.ds(i, L)] = zi

    def _fire(off, kb, qb, sem):
        pltpu.make_async_copy(k_hbm.at[srca.at[pl.ds(off, W_A)]], kb,
                              sem).start()
        pltpu.make_async_copy(q_hbm.at[dsta.at[pl.ds(off, W_A)]], qb,
                              sem).start()

    def _wait(off, kb, qb, sem):
        pltpu.make_async_copy(k_hbm.at[srca.at[pl.ds(off, W_A)]], kb,
                              sem).wait()
        pltpu.make_async_copy(q_hbm.at[dsta.at[pl.ds(off, W_A)]], qb,
                              sem).wait()

    def _process(off, kb, qb):
        @pl.loop(0, W_A, step=L)
        def _(g):
            ev = jnp.zeros((L,), jnp.float32)
            for j in range(L):
                i = g + j
                acc = kb[i, pl.ds(0, L)] * qb[i, pl.ds(0, L)]
                for t in range(1, D // L):
                    acc = acc + kb[i, pl.ds(t * L, L)] * qb[i, pl.ds(t * L, L)]
                ev = jnp.where(lane == j, jnp.sum(acc), ev)
            ebig[pl.ds(off + g, L)] = ev
            dvec = dsta[pl.ds(off + g, L)]

            def _mbody(_):
                cur = plsc.load_gather(m_part, [dvec])
                pend = cur < ev
                plsc.store_scatter(m_part, [dvec], jnp.maximum(cur, ev),
                                   mask=pend)
                return jnp.any(pend)

            lax.while_loop(lambda go: go, _mbody, jnp.bool_(True))

    _fire(0, kb0, qb0, sem0)

    @pl.loop(0, NCH_A, step=2)
    def _(ci):
        off0 = ci * W_A
        _fire(off0 + W_A, kb1, qb1, sem1)
        _wait(off0, kb0, qb0, sem0)
        _process(off0, kb0, qb0)
        _fire(off0 + 2 * W_A, kb0, qb0, sem0)
        _wait(off0 + W_A, kb1, qb1, sem1)
        _process(off0 + W_A, kb1, qb1)

    _wait(NCH_A * W_A, kb0, qb0, sem0)

    pltpu.sync_copy(ebig, e_hbm.at[pl.ds(base, EPT_A)])
    pltpu.sync_copy(m_part, mpart_hbm.at[pl.ds(w * N_PAD, N_PAD)])


@functools.partial(
    pl.kernel,
    out_type=jax.ShapeDtypeStruct((N_PAD, D), jnp.float32),
    mesh=_mesh,
    compiler_params=_sc_params,
    scratch_types=[
        pltpu.VMEM((RPT, D), jnp.float32),
        pltpu.VMEM((RPT,), jnp.float32),
        pltpu.VMEM((RPT,), jnp.float32),
        pltpu.VMEM((RPT,), jnp.float32),
        pltpu.VMEM((RPT,), jnp.float32),
        pltpu.VMEM((W_B,), jnp.int32),
        pltpu.VMEM((W_B,), jnp.int32),
        pltpu.VMEM((W_B,), jnp.int32),
        pltpu.VMEM((W_B,), jnp.int32),
        pltpu.VMEM((W_B,), jnp.int32),
        pltpu.VMEM((W_B,), jnp.int32),
        pltpu.VMEM((W_B,), jnp.int32),
        pltpu.VMEM((W_B,), jnp.int32),
        pltpu.VMEM((W_B,), jnp.float32),
        pltpu.VMEM((W_B,), jnp.float32),
        pltpu.VMEM((W_B,), jnp.float32),
        pltpu.VMEM((W_B,), jnp.float32),
        pltpu.VMEM((SB, D), jnp.float32),
        pltpu.VMEM((SB, D), jnp.float32),
        pltpu.SemaphoreType.DMA,
        pltpu.SemaphoreType.DMA,
        pltpu.SemaphoreType.DMA,
        pltpu.SemaphoreType.DMA,
        pltpu.SemaphoreType.DMA,
        pltpu.SemaphoreType.DMA,
    ],
)
def _aggregate(v_hbm, src_hbm, dst_hbm, e_hbm, mpart_hbm, h_hbm,
               h_own, den_own, m_own, m_st0, m_st1,
               sb0, sb1, sb2, sb3, db0, db1, db2, db3, ebf0, ebf1, ebf2,
               ebf3, vg0, vg1, sm0, sm1, sm2, sm3, smv0, smv1):
    c = lax.axis_index("c")
    s = lax.axis_index("s")
    w = s * NC + c
    r0 = w * RPT
    lane = lax.iota(jnp.int32, L)
    zv = jnp.zeros((L,), jnp.float32)

    def _mfire(p, st, sem):
        poff = pl.multiple_of(p * N_PAD + r0, 8)
        pltpu.make_async_copy(mpart_hbm.at[pl.ds(poff, RPT)], st,
                              sem).start()

    def _mwait(p, st, sem):
        poff = pl.multiple_of(p * N_PAD + r0, 8)
        pltpu.make_async_copy(mpart_hbm.at[pl.ds(poff, RPT)], st, sem).wait()

    def _mmax(st):
        @pl.loop(0, RPT, step=L)
        def _(i):
            m_own[pl.ds(i, L)] = jnp.maximum(m_own[pl.ds(i, L)],
                                             st[pl.ds(i, L)])

    pltpu.sync_copy(mpart_hbm.at[pl.ds(pl.multiple_of(r0, 8), RPT)], m_own)
    _mfire(1, m_st0, sm0)

    @pl.loop(1, NW - 1, step=2)
    def _(p):
        _mfire(p + 1, m_st1, sm1)
        _mwait(p, m_st0, sm0)
        _mmax(m_st0)
        _mfire(p + 2, m_st0, sm0)
        _mwait(p + 1, m_st1, sm1)
        _mmax(m_st1)

    _mwait(NW - 1, m_st0, sm0)
    _mmax(m_st0)

    @pl.loop(0, RPT)
    def _(r):
        for t in range(D // L):
            h_own[r, pl.ds(t * L, L)] = zv

    @pl.loop(0, RPT, step=L)
    def _(i):
        den_own[pl.ds(i, L)] = zv

    def _sfire(ch, sb, db, eb, sem):
        off = ch * W_B
        pltpu.make_async_copy(src_hbm.at[pl.ds(off, W_B)], sb, sem).start()
        pltpu.make_async_copy(dst_hbm.at[pl.ds(off, W_B)], db, sem).start()
        pltpu.make_async_copy(e_hbm.at[pl.ds(off, W_B)], eb, sem).start()

    def _swait(ch, sb, db, eb, sem):
        off = ch * W_B
        pltpu.make_async_copy(src_hbm.at[pl.ds(off, W_B)], sb, sem).wait()
        pltpu.make_async_copy(dst_hbm.at[pl.ds(off, W_B)], db, sem).wait()
        pltpu.make_async_copy(e_hbm.at[pl.ds(off, W_B)], eb, sem).wait()

    def _compact(sb, db, eb):
        def _body(gi, np_):
            gg = gi * L
            dl = db[pl.ds(gg, L)] - r0
            valid = (dl >= 0) & (dl < RPT)
            plsc.store_compressed(sb.at[pl.ds(np_, L)],
                                  sb[pl.ds(gg, L)], mask=valid)
            plsc.store_compressed(db.at[pl.ds(np_, L)], dl, mask=valid)
            plsc.store_compressed(eb.at[pl.ds(np_, L)],
                                  eb[pl.ds(gg, L)], mask=valid)
            cnt = plsc.all_reduce_population_count(valid)
            return np_ + cnt[0]

        return lax.fori_loop(0, W_B // L, _body, jnp.int32(0))

    def _vfire(sb, vg, sem, np_):
        ng16 = jnp.minimum(lax.shift_right_logical(np_ + (L - 1), 4),
                           SB // L)

        @pl.loop(0, ng16)
        def _(g):
            pltpu.make_async_copy(v_hbm.at[sb.at[pl.ds(g * L, L)]],
                                  vg.at[pl.ds(g * L, L)], sem).start()

    def _vwait(sb, vg, sem, np_):
        ng16 = jnp.minimum(lax.shift_right_logical(np_ + (L - 1), 4),
                           SB // L)

        @pl.loop(0, ng16)
        def _(g):
            pltpu.make_async_copy(v_hbm.at[sb.at[pl.ds(g * L, L)]],
                                  vg.at[pl.ds(g * L, L)], sem).wait()

    def _accum_group(db, eb, vg, gj, vbase, np_):
        gbase = gj * L
        dlg = db[pl.ds(gbase, L)]
        eg = eb[pl.ds(gbase, L)]
        validv = (gbase + lane) < np_
        dls = jnp.where(validv, dlg, 0)
        mg = plsc.load_gather(m_own, [dls])
        exv = jnp.where(validv, jnp.exp(eg - mg), 0.0)
        for j in range(L):
            dj = dls[j]
            exj = exv[j]
            vrow = vbase + j
            for t in range(D // L):
                plsc.addupdate(h_own.at[dj, pl.ds(t * L, L)],
                               vg[vrow, pl.ds(t * L, L)] * exj)
            plsc.addupdate_scatter(
                den_own, [jnp.full_like(lane, dj)],
                jnp.where(lane == 0, exj, 0.0), mask=lane == 0)

    def _accum(sb, db, eb, vg, np_):
        ngj = lax.shift_right_logical(np_ + (L - 1), 4)

        @pl.loop(0, ngj)
        def _(gj):
            inbuf = gj < (SB // L)

            @pl.when(jnp.logical_not(inbuf))
            def _():
                pltpu.sync_copy(v_hbm.at[sb.at[pl.ds(gj * L, L)]],
                                vg.at[pl.ds(0, L)])

            vbase = jnp.where(inbuf, gj * L, 0)
            _accum_group(db, eb, vg, gj, vbase, np_)

    _ring = [(sb0, db0, ebf0, sm0), (sb1, db1, ebf1, sm1),
             (sb2, db2, ebf2, sm2), (sb3, db3, ebf3, sm3)]
    _vring = [(vg0, smv0), (vg1, smv1)]

    _sfire(0, sb0, db0, ebf0, sm0)
    _sfire(1, sb1, db1, ebf1, sm1)
    _sfire(2, sb2, db2, ebf2, sm2)

    def _stage(k, np_prev, pos):
        sbk, dbk, ebk, smk = _ring[pos % 4]
        vgk, smvk = _vring[pos % 2]
        sbp, dbp, ebp, smp = _ring[(pos - 1) % 4]
        vgp, smvp = _vring[(pos - 1) % 2]
        _swait(k, sbk, dbk, ebk, smk)
        np_k = _compact(sbk, dbk, ebk)
        _vfire(sbk, vgk, smvk, np_k)
        _vwait(sbp, vgp, smvp, np_prev)
        _accum(sbp, dbp, ebp, vgp, np_prev)

        @pl.when(k + 3 < NCH_B)
        def _():
            _sfire(k + 3, sbp, dbp, ebp, smp)

        return np_k

    def _quad(it, np_carry):
        k = it * 4
        np_carry = _stage(k, np_carry, 0)
        np_carry = _stage(k + 1, np_carry, 1)
        np_carry = _stage(k + 2, np_carry, 2)
        np_carry = _stage(k + 3, np_carry, 3)
        return np_carry

    np_last = lax.fori_loop(0, NCH_B // 4, _quad, jnp.int32(0))

    _vwait(sb3, vg1, smv1, np_last)
    _accum(sb3, db3, ebf3, vg1, np_last)

    @pl.loop(0, RPT, step=16)
    def _(r):
        denv = den_own[pl.ds(r, 16)]
        scale16 = jnp.where(denv > 0, 1.0 / jnp.where(denv > 0, denv, 1.0),
                            0.0)
        for j in range(16):
            scale = scale16[j]
            for t in range(D // L):
                h_own[r + j, pl.ds(t * L, L)] = (h_own[r + j, pl.ds(t * L, L)]
                                                 * scale)
        pltpu.make_async_copy(h_own.at[pl.ds(r, 16)],
                              h_hbm.at[pl.ds(r0 + r, 16)], sm0).start()

    @pl.loop(0, RPT, step=16)
    def _(r):
        pltpu.make_async_copy(h_own.at[pl.ds(r, 16)],
                              h_hbm.at[pl.ds(r0 + r, 16)], sm0).wait()


def kernel(z, edge_index, Wq, bq, Wk, bk, Wv, bv):
    ei = edge_index.astype(jnp.int32)
    src, dst = ei[0], ei[1]
    pad = E_ALLOC - E
    src_p = jnp.concatenate([src, jnp.zeros((pad,), jnp.int32)])
    dst_p = jnp.concatenate([dst, jnp.full((pad,), N_PAD - 1, jnp.int32)])
    q, k, v = _proj(z, Wq.T, Wk.T, Wv.T,
                    bq.reshape(1, D), bk.reshape(1, D), bv.reshape(1, D))
    e, mpart = _edge_logits(k, q, src_p, dst_p)
    h = _aggregate(v, src_p, dst_p, e, mpart)
    return h[:N]
